# baseline (speedup 1.0000x reference)
import functools

import jax
import jax.numpy as jnp
from jax import lax
from jax.experimental import pallas as pl
from jax.experimental.pallas import tpu as pltpu
from jax.experimental.pallas import tpu_sc as plsc


@functools.lru_cache(maxsize=None)
def _make_line_gather(batch: int, hist: int, dim: int):
    info = plsc.get_sparse_core_info()
    nc, ns = info.num_cores, info.num_subcores
    nw = nc * ns
    r_per_w = batch // nw
    assert r_per_w * nw == batch
    nr = 8
    while r_per_w % (2 * nr):
        nr //= 2
    n_chunks = r_per_w // nr
    n_outer = n_chunks // 2
    line = 2 * dim

    mesh = plsc.VectorSubcoreMesh(core_axis_name="c", subcore_axis_name="s")

    @functools.partial(
        pl.kernel,
        mesh=mesh,
        out_type=jax.ShapeDtypeStruct((batch, hist, line), jnp.float32),
        scratch_types=[
            pltpu.VMEM((r_per_w, hist), jnp.int32),
            pltpu.VMEM((2, nr, hist, line), jnp.float32),
            pltpu.SemaphoreType.DMA,
            pltpu.SemaphoreType.DMA,
        ],
        compiler_params=pltpu.CompilerParams(use_tc_tiling_on_sc=True),
    )
    def line_gather(idx_hbm, table_hbm, out_hbm, idx_v, rows_v, sem0, sem1):
        wid = lax.axis_index("s") * nc + lax.axis_index("c")
        base = wid * r_per_w
        sems = (sem0, sem1)

        pltpu.sync_copy(idx_hbm.at[pl.ds(base, r_per_w)], idx_v)

        def fire(c, b):
            def one_row(r, carry):
                pltpu.async_copy(
                    table_hbm.at[idx_v.at[c * nr + r, :]],
                    rows_v.at[b, r],
                    sems[b],
                )
                return carry

            lax.fori_loop(0, nr, one_row, 0)

        def drain(c, b):
            def one_row(r, carry):
                pltpu.make_async_copy(
                    table_hbm.at[idx_v.at[c * nr + r, :]],
                    rows_v.at[b, r],
                    sems[b],
                ).wait()
                return carry

            lax.fori_loop(0, nr, one_row, 0)
            pltpu.sync_copy(
                rows_v.at[b], out_hbm.at[pl.ds(base + c * nr, nr)]
            )

        fire(0, 0)

        def body(step, carry):
            c0 = step * 2
            fire(c0 + 1, 1)
            drain(c0, 0)

            @pl.when(step < n_outer - 1)
            def _():
                fire(c0 + 2, 0)

            drain(c0 + 1, 1)
            return carry

        lax.fori_loop(0, n_outer, body, 0)

    return line_gather


def _extract_block(lines_ref, idx_ref, out_ref):
    hist = out_ref.shape[1]
    dim = out_ref.shape[2]
    for h in range(hist):
        par = (idx_ref[:, h] & 1)[:, None]
        lo = lines_ref[:, h, :dim]
        hi = lines_ref[:, h, dim:]
        out_ref[:, h, :] = jnp.where(par == 1, hi, lo)


@functools.lru_cache(maxsize=None)
def _make_extract(batch: int, hist: int, dim: int):
    bb = 512
    line = 2 * dim
    return pl.pallas_call(
        _extract_block,
        grid=(batch // bb,),
        in_specs=[
            pl.BlockSpec((bb, hist, line), lambda i: (i, 0, 0)),
            pl.BlockSpec((bb, hist), lambda i: (i, 0)),
        ],
        out_specs=pl.BlockSpec((bb, hist, dim), lambda i: (i, 0, 0)),
        out_shape=jax.ShapeDtypeStruct((batch, hist, dim), jnp.float32),
    )


def kernel(input, weight):
    b, h = input.shape
    dim = weight.shape[1]
    idx = input.astype(jnp.int32)
    lines = _make_line_gather(b, h, dim)(
        idx >> 1, weight.reshape(weight.shape[0] // 2, 2 * dim)
    )
    return _make_extract(b, h, dim)(lines, idx)


# speedup vs baseline: 1.0271x; 1.0271x over previous
import functools

import jax
import jax.numpy as jnp
from jax import lax
from jax.experimental import pallas as pl
from jax.experimental.pallas import tpu as pltpu
from jax.experimental.pallas import tpu_sc as plsc


@functools.lru_cache(maxsize=None)
def _make_line_gather(batch: int, hist: int, dim: int):
    info = plsc.get_sparse_core_info()
    nc, ns = info.num_cores, info.num_subcores
    nw = nc * ns
    r_per_w = batch // nw
    assert r_per_w * nw == batch
    nr = 8
    while r_per_w % (2 * nr):
        nr //= 2
    n_chunks = r_per_w // nr
    n_outer = n_chunks // 2
    line = 2 * dim

    mesh = plsc.VectorSubcoreMesh(core_axis_name="c", subcore_axis_name="s")

    @functools.partial(
        pl.kernel,
        mesh=mesh,
        out_type=jax.ShapeDtypeStruct((batch, hist, line), jnp.float32),
        scratch_types=[
            pltpu.VMEM((r_per_w, hist), jnp.int32),
            pltpu.VMEM((2, nr, hist, line), jnp.float32),
            pltpu.SemaphoreType.DMA,
            pltpu.SemaphoreType.DMA,
        ],
        compiler_params=pltpu.CompilerParams(use_tc_tiling_on_sc=True),
    )
    def line_gather(idx_hbm, table_hbm, out_hbm, idx_v, rows_v, sem0, sem1):
        wid = lax.axis_index("s") * nc + lax.axis_index("c")
        base = wid * r_per_w
        sems = (sem0, sem1)

        pltpu.sync_copy(idx_hbm.at[pl.ds(base, r_per_w)], idx_v)

        def fire(c, b):
            def one_row(r, carry):
                pltpu.async_copy(
                    table_hbm.at[idx_v.at[c * nr + r, :]],
                    rows_v.at[b, r],
                    sems[b],
                )
                return carry

            lax.fori_loop(0, nr, one_row, 0)

        def drain(c, b):
            def one_row(r, carry):
                pltpu.make_async_copy(
                    table_hbm.at[idx_v.at[c * nr + r, :]],
                    rows_v.at[b, r],
                    sems[b],
                ).wait()
                return carry

            lax.fori_loop(0, nr, one_row, 0)
            pltpu.sync_copy(
                rows_v.at[b], out_hbm.at[pl.ds(base + c * nr, nr)]
            )

        fire(0, 0)

        def body(step, carry):
            c0 = step * 2
            fire(c0 + 1, 1)
            drain(c0, 0)

            @pl.when(step < n_outer - 1)
            def _():
                fire(c0 + 2, 0)

            drain(c0 + 1, 1)
            return carry

        lax.fori_loop(0, n_outer, body, 0)

    return line_gather


def _extract_block(lines_ref, idx_ref, out_ref):
    dim = out_ref.shape[2]
    lines = lines_ref[...]
    parf = (idx_ref[...] & 1).astype(jnp.float32)
    lo = lines[:, :, :dim]
    hi = lines[:, :, dim:]
    out_ref[...] = lo + (hi - lo) * parf[:, :, None]


@functools.lru_cache(maxsize=None)
def _make_extract(batch: int, hist: int, dim: int):
    bb = 512
    line = 2 * dim
    return pl.pallas_call(
        _extract_block,
        grid=(batch // bb,),
        in_specs=[
            pl.BlockSpec((bb, hist, line), lambda i: (i, 0, 0)),
            pl.BlockSpec((bb, hist), lambda i: (i, 0)),
        ],
        out_specs=pl.BlockSpec((bb, hist, dim), lambda i: (i, 0, 0)),
        out_shape=jax.ShapeDtypeStruct((batch, hist, dim), jnp.float32),
    )


def kernel(input, weight):
    b, h = input.shape
    dim = weight.shape[1]
    idx = input.astype(jnp.int32)
    lines = _make_line_gather(b, h, dim)(
        idx >> 1, weight.reshape(weight.shape[0] // 2, 2 * dim)
    )
    return _make_extract(b, h, dim)(lines, idx)
